# baseline (speedup 1.0000x reference)
import functools

import jax
import jax.numpy as jnp
from jax import lax
from jax.experimental import pallas as pl
from jax.experimental.pallas import tpu as pltpu
from jax.experimental.pallas import tpu_sc as plsc

B = 16
MAX_LEN = 4096
D = 1024
NW = 32
ROWS_PER_W = (B * MAX_LEN) // NW
UNIT = 64
NUNITS = ROWS_PER_W // UNIT
RING = 8


def _pad_body(flat_hbm, params_hbm, zeros_hbm, out_hbm, pvec, zbuf, sem):
    wid = lax.axis_index("s") * 2 + lax.axis_index("c")

    pltpu.sync_copy(params_hbm.at[pl.ds(wid * 16, 16)], pvec)
    pltpu.sync_copy(zeros_hbm, zbuf)

    pv = pvec[...]
    start = pv[0]
    valid = pv[1]
    outbase = wid * ROWS_PER_W

    pending = []

    def drain_one():
        pred, dcopy, dzero = pending.pop(0)

        @pl.when(pred)
        def _wait_copy():
            dcopy.wait()

        @pl.when(jnp.logical_not(pred))
        def _wait_zero():
            dzero.wait()

    for u in range(NUNITS):
        full_copy = valid >= (u + 1) * UNIT
        dcopy = pltpu.make_async_copy(
            flat_hbm.at[pl.ds((start + u * UNIT) * D, UNIT * D)],
            out_hbm.at[pl.ds((outbase + u * UNIT) * D, UNIT * D)],
            sem,
        )
        dzero = pltpu.make_async_copy(
            zbuf,
            out_hbm.at[pl.ds((outbase + u * UNIT) * D, UNIT * D)],
            sem,
        )

        @pl.when(full_copy)
        def _copy_unit(dcopy=dcopy):
            dcopy.start()

        @pl.when(jnp.logical_not(full_copy))
        def _zero_unit(dzero=dzero):
            dzero.start()

        pending.append((full_copy, dcopy, dzero))
        if len(pending) > RING:
            drain_one()
    while pending:
        drain_one()

    u0 = valid // UNIT
    p = valid - u0 * UNIT

    @pl.when(p > 0)
    def _patch():
        src0 = start + u0 * UNIT
        dst0 = outbase + u0 * UNIT
        patch_pending = []

        consumed = jnp.int32(0)
        for k in (5, 4, 3, 2, 1, 0):
            sz = 1 << k
            take = ((p >> k) & 1) == 1
            desc = pltpu.make_async_copy(
                flat_hbm.at[pl.ds((src0 + consumed) * D, sz * D)],
                out_hbm.at[pl.ds((dst0 + consumed) * D, sz * D)],
                sem,
            )

            @pl.when(take)
            def _copy_bit(desc=desc):
                desc.start()

            patch_pending.append((take, desc))
            consumed = consumed + jnp.where(take, sz, 0)

        q = UNIT - p
        zconsumed = jnp.int32(0)
        for k in (5, 4, 3, 2, 1, 0):
            sz = 1 << k
            take = ((q >> k) & 1) == 1
            desc = pltpu.make_async_copy(
                zbuf.at[pl.ds(0, sz * D)],
                out_hbm.at[pl.ds((dst0 + p + zconsumed) * D, sz * D)],
                sem,
            )

            @pl.when(take)
            def _zero_bit(desc=desc):
                desc.start()

            patch_pending.append((take, desc))
            zconsumed = zconsumed + jnp.where(take, sz, 0)

        for take, desc in patch_pending:

            @pl.when(take)
            def _wait_bit(desc=desc):
                desc.wait()


_pad_kernel = functools.partial(
    pl.kernel,
    out_type=jax.ShapeDtypeStruct((B * MAX_LEN * D,), jnp.float32),
    mesh=plsc.VectorSubcoreMesh(core_axis_name="c", subcore_axis_name="s"),
    scratch_types=[
        pltpu.VMEM((16,), jnp.int32),
        pltpu.VMEM((UNIT * D,), jnp.float32),
        pltpu.SemaphoreType.DMA,
    ],
)(_pad_body)


def kernel(flat, cu_seqlens):
    cu = cu_seqlens.astype(jnp.int32)
    lens32 = cu[1:] - cu[:-1]

    w = jnp.arange(NW, dtype=jnp.int32)
    b = w // 2
    t0 = (w % 2) * ROWS_PER_W
    starts = cu[:-1][b] + t0
    valids = jnp.clip(lens32[b] - t0, 0, ROWS_PER_W)
    params = jnp.zeros((NW, 16), jnp.int32)
    params = params.at[:, 0].set(starts).at[:, 1].set(valids)

    zeros = jnp.zeros((UNIT * D,), jnp.float32)
    out = _pad_kernel(flat.reshape(-1), params.reshape(-1), zeros)
    padded = out.reshape(B, MAX_LEN, D)
    lens = lens32.astype(jnp.int64)
    return padded, lens


# speedup vs baseline: 5.6858x; 5.6858x over previous
import functools

import jax
import jax.numpy as jnp
from jax import lax
from jax.experimental import pallas as pl
from jax.experimental.pallas import tpu as pltpu
from jax.experimental.pallas import tpu_sc as plsc

B = 16
MAX_LEN = 4096
D = 1024
NW = 32
ROWS_PER_W = (B * MAX_LEN) // NW
UNIT = 32
NUNITS = ROWS_PER_W // UNIT
ZRING = 8


def _pad_body(
    flat_hbm, params_hbm, zeros_hbm, out_hbm, pvec, zbuf, vb0, vb1, gsem, ssem, zsem
):
    wid = lax.axis_index("s") * 2 + lax.axis_index("c")

    pltpu.sync_copy(params_hbm.at[pl.ds(wid * 16, 16)], pvec)
    pltpu.sync_copy(zeros_hbm, zbuf)

    pv = pvec[...]
    start = pv[0]
    valid = pv[1]
    outbase = wid * ROWS_PER_W

    bufs = (vb0, vb1)
    units = []

    for u in range(NUNITS):
        is_copy = valid >= (u + 1) * UNIT
        buf = bufs[u % 2]
        src = (start + u * UNIT) * D
        dst = (outbase + u * UNIT) * D
        gd = pltpu.make_async_copy(flat_hbm.at[pl.ds(src, UNIT * D)], buf, gsem)
        sd = pltpu.make_async_copy(buf, out_hbm.at[pl.ds(dst, UNIT * D)], ssem)
        zd = pltpu.make_async_copy(zbuf, out_hbm.at[pl.ds(dst, UNIT * D)], zsem)

        if u >= 2:
            pred2, _, sd2, _ = units[u - 2]

            @pl.when(pred2)
            def _wait_scatter(sd2=sd2):
                sd2.wait()

        if u >= 1:
            pred1, gd1, sd1, _ = units[u - 1]

            @pl.when(pred1)
            def _advance(gd1=gd1, sd1=sd1):
                gd1.wait()
                sd1.start()

        @pl.when(is_copy)
        def _start_gather(gd=gd):
            gd.start()

        @pl.when(jnp.logical_not(is_copy))
        def _start_zero(zd=zd):
            zd.start()

        units.append((is_copy, gd, sd, zd))

        if u >= ZRING:
            predz, _, _, zdz = units[u - ZRING]

            @pl.when(jnp.logical_not(predz))
            def _wait_zero(zdz=zdz):
                zdz.wait()

    predl, gdl, sdl, _ = units[NUNITS - 1]

    @pl.when(predl)
    def _advance_last():
        gdl.wait()
        sdl.start()

    for u in (NUNITS - 2, NUNITS - 1):
        predu, _, sdu, _ = units[u]

        @pl.when(predu)
        def _wait_scatter_tail(sdu=sdu):
            sdu.wait()

    for u in range(NUNITS - ZRING, NUNITS):
        predu, _, _, zdu = units[u]

        @pl.when(jnp.logical_not(predu))
        def _wait_zero_tail(zdu=zdu):
            zdu.wait()

    u0 = valid // UNIT
    p = valid - u0 * UNIT

    @pl.when(p > 0)
    def _patch():
        src0 = start + u0 * UNIT
        dst0 = outbase + u0 * UNIT

        consumed = jnp.int32(0)
        for k in (4, 3, 2, 1, 0):
            sz = 1 << k
            take = ((p >> k) & 1) == 1
            gd = pltpu.make_async_copy(
                flat_hbm.at[pl.ds((src0 + consumed) * D, sz * D)],
                vb0.at[pl.ds(0, sz * D)],
                gsem,
            )
            sd = pltpu.make_async_copy(
                vb0.at[pl.ds(0, sz * D)],
                out_hbm.at[pl.ds((dst0 + consumed) * D, sz * D)],
                ssem,
            )

            @pl.when(take)
            def _copy_bit(gd=gd, sd=sd):
                gd.start()
                gd.wait()
                sd.start()
                sd.wait()

            consumed = consumed + jnp.where(take, sz, 0)

        q = UNIT - p
        zconsumed = jnp.int32(0)
        for k in (4, 3, 2, 1, 0):
            sz = 1 << k
            take = ((q >> k) & 1) == 1
            zd = pltpu.make_async_copy(
                zbuf.at[pl.ds(0, sz * D)],
                out_hbm.at[pl.ds((dst0 + p + zconsumed) * D, sz * D)],
                zsem,
            )

            @pl.when(take)
            def _zero_bit(zd=zd):
                zd.start()
                zd.wait()

            zconsumed = zconsumed + jnp.where(take, sz, 0)


_pad_kernel = functools.partial(
    pl.kernel,
    out_type=jax.ShapeDtypeStruct((B * MAX_LEN * D,), jnp.float32),
    mesh=plsc.VectorSubcoreMesh(core_axis_name="c", subcore_axis_name="s"),
    scratch_types=[
        pltpu.VMEM((16,), jnp.int32),
        pltpu.VMEM((UNIT * D,), jnp.float32),
        pltpu.VMEM((UNIT * D,), jnp.float32),
        pltpu.VMEM((UNIT * D,), jnp.float32),
        pltpu.SemaphoreType.DMA,
        pltpu.SemaphoreType.DMA,
        pltpu.SemaphoreType.DMA,
    ],
)(_pad_body)


def kernel(flat, cu_seqlens):
    cu = cu_seqlens.astype(jnp.int32)
    lens32 = cu[1:] - cu[:-1]

    w = jnp.arange(NW, dtype=jnp.int32)
    b = w // 2
    t0 = (w % 2) * ROWS_PER_W
    starts = cu[:-1][b] + t0
    valids = jnp.clip(lens32[b] - t0, 0, ROWS_PER_W)
    params = jnp.zeros((NW, 16), jnp.int32)
    params = params.at[:, 0].set(starts).at[:, 1].set(valids)

    zeros = jnp.zeros((UNIT * D,), jnp.float32)
    out = _pad_kernel(flat.reshape(-1), params.reshape(-1), zeros)
    padded = out.reshape(B, MAX_LEN, D)
    lens = lens32.astype(jnp.int64)
    return padded, lens


# speedup vs baseline: 20.9484x; 3.6844x over previous
import functools

import jax
import jax.numpy as jnp
from jax import lax
from jax.experimental import pallas as pl
from jax.experimental.pallas import tpu as pltpu
from jax.experimental.pallas import tpu_sc as plsc

B = 16
MAX_LEN = 4096
D = 1024
NW = 32
ROWS_PER_W = (B * MAX_LEN) // NW
UNIT = 32
NUNITS = ROWS_PER_W // UNIT
ZROWS = 16
ZRING = 8
TOTAL_ROWS = B * MAX_LEN


def _build_kernel():
    mesh = plsc.VectorSubcoreMesh(core_axis_name="c", subcore_axis_name="s")

    def body(
        flat_hbm,
        params_hbm,
        zeros_hbm,
        out_hbm,
        pvec,
        zbuf,
        idx_all,
        vb0,
        vb1,
        vb2,
        gsem,
        ssem,
        zsem,
        lsem,
    ):
        wid = lax.axis_index("s") * 2 + lax.axis_index("c")

        pltpu.sync_copy(params_hbm.at[pl.ds(wid * 16, 16)], pvec)
        pltpu.sync_copy(zeros_hbm, zbuf)

        pv = pvec[...]
        start = pv[0]
        valid = pv[1]
        outbase = wid * ROWS_PER_W

        lane = lax.broadcasted_iota(jnp.int32, (16,), 0)

        def idx_body(i, carry):
            idx_all[pl.ds(i * 16, 16)] = jnp.minimum(
                start + i * 16 + lane, TOTAL_ROWS - 1
            )
            return carry

        lax.fori_loop(0, ROWS_PER_W // 16, idx_body, 0)

        u0 = valid // UNIT
        p = valid - u0 * UNIT

        def advance(prev_pred, prev_gd, prev_sd):
            @pl.when(prev_pred)
            def _():
                prev_gd.wait()
                prev_sd.start()

        bufs = (vb0, vb1, vb2)
        units = []

        for u in range(NUNITS):
            is_copy = valid >= (u + 1) * UNIT
            buf = bufs[u % 3]
            dst = outbase + u * UNIT
            gd = pltpu.make_async_copy(
                flat_hbm.at[idx_all.at[pl.ds(u * UNIT, UNIT)]], buf, gsem
            )
            sd = pltpu.make_async_copy(buf, out_hbm.at[pl.ds(dst, UNIT)], ssem)
            zda = pltpu.make_async_copy(
                zbuf, out_hbm.at[pl.ds(dst, ZROWS)], zsem
            )
            zdb = pltpu.make_async_copy(
                zbuf, out_hbm.at[pl.ds(dst + ZROWS, ZROWS)], zsem
            )

            if u >= 3:
                pred3, _, sd3, _, _ = units[u - 3]

                @pl.when(pred3)
                def _wait_scatter(sd3=sd3):
                    sd3.wait()

            @pl.when(is_copy)
            def _start_gather(gd=gd):
                gd.start()

            @pl.when(jnp.logical_not(is_copy))
            def _start_zero(zda=zda, zdb=zdb):
                zda.start()
                zdb.start()

            if u >= 1:
                pu = units[u - 1]
                advance(pu[0], pu[1], pu[2])

            units.append((is_copy, gd, sd, zda, zdb))

            if u >= ZRING:
                predz, _, _, za, zb = units[u - ZRING]

                @pl.when(jnp.logical_not(predz))
                def _wait_zero(za=za, zb=zb):
                    za.wait()
                    zb.wait()

        pu = units[NUNITS - 1]
        advance(pu[0], pu[1], pu[2])
        for u in (NUNITS - 3, NUNITS - 2, NUNITS - 1):
            predu, _, sdu, _, _ = units[u]

            @pl.when(predu)
            def _wait_scatter_tail(sdu=sdu):
                sdu.wait()

        for u in range(NUNITS - ZRING, NUNITS):
            predu, _, _, za, zb = units[u]

            @pl.when(jnp.logical_not(predu))
            def _wait_zero_tail(za=za, zb=zb):
                za.wait()
                zb.wait()

        @pl.when(p > 0)
        def _straddle():
            gd = pltpu.make_async_copy(
                flat_hbm.at[
                    idx_all.at[pl.ds(pl.multiple_of(u0 * UNIT, UNIT), UNIT)]
                ],
                vb0,
                gsem,
            )
            gd.start()
            gd.wait()

            zero16 = jnp.zeros((16,), jnp.float32)

            def zrow(i, carry):
                r = p + i
                for c in range(D // 16):
                    vb0[r, pl.ds(c * 16, 16)] = zero16
                return carry

            lax.fori_loop(0, UNIT - p, zrow, 0)

            sd = pltpu.make_async_copy(
                vb0,
                out_hbm.at[
                    pl.ds(pl.multiple_of(outbase + u0 * UNIT, UNIT), UNIT)
                ],
                ssem,
            )
            sd.start()
            sd.wait()

    return functools.partial(
        pl.kernel,
        out_type=jax.ShapeDtypeStruct((B * MAX_LEN, D), jnp.float32),
        mesh=mesh,
        scratch_types=[
            pltpu.VMEM((16,), jnp.int32),
            pltpu.VMEM((ZROWS, D), jnp.float32),
            pltpu.VMEM((ROWS_PER_W,), jnp.int32),
            pltpu.VMEM((UNIT, D), jnp.float32),
            pltpu.VMEM((UNIT, D), jnp.float32),
            pltpu.VMEM((UNIT, D), jnp.float32),
            pltpu.SemaphoreType.DMA,
            pltpu.SemaphoreType.DMA,
            pltpu.SemaphoreType.DMA,
            pltpu.SemaphoreType.DMA,
        ],
    )(body)


_pad_kernel = _build_kernel()


def kernel(flat, cu_seqlens):
    cu = cu_seqlens.astype(jnp.int32)
    lens32 = cu[1:] - cu[:-1]

    w = jnp.arange(NW, dtype=jnp.int32)
    b = w // 2
    t0 = (w % 2) * ROWS_PER_W
    starts = cu[:-1][b] + t0
    valids = jnp.clip(lens32[b] - t0, 0, ROWS_PER_W)
    params = jnp.zeros((NW, 16), jnp.int32)
    params = params.at[:, 0].set(starts).at[:, 1].set(valids)

    zeros = jnp.zeros((ZROWS, D), jnp.float32)
    out = _pad_kernel(flat, params.reshape(-1), zeros)
    padded = out.reshape(B, MAX_LEN, D)
    lens = lens32.astype(jnp.int64)
    return padded, lens
